# Initial kernel scaffold; baseline (speedup 1.0000x reference)
import jax
import jax.numpy as jnp
from jax.experimental import pallas as pl


def kernel(in0, in1):
    raise NotImplementedError("write your pallas kernel here")


# speedup vs baseline: 9.1410x; 9.1410x over previous
import jax
import jax.numpy as jnp
from jax.experimental import pallas as pl

_NUM_SEG = 8
_EXTENT = 64
_FEAT = _NUM_SEG * _EXTENT


def _conv_kernel(x0_ref, x1_ref, o_ref):
    x0 = x0_ref[...]
    x1 = x1_ref[...]
    s0 = [x0[:, i * _EXTENT:(i + 1) * _EXTENT] for i in range(_NUM_SEG)]
    s1 = [x1[:, j * _EXTENT:(j + 1) * _EXTENT] for j in range(_NUM_SEG)]
    for k in range(_NUM_SEG):
        acc = s0[0] * s1[k]
        for i in range(1, _NUM_SEG):
            acc = acc + s0[i] * s1[(k - i) % _NUM_SEG]
        o_ref[:, k * _EXTENT:(k + 1) * _EXTENT] = acc


def kernel(in0, in1):
    B = in0.shape[0]
    BB = 1024
    return pl.pallas_call(
        _conv_kernel,
        grid=(B // BB,),
        in_specs=[
            pl.BlockSpec((BB, _FEAT), lambda i: (i, 0)),
            pl.BlockSpec((BB, _FEAT), lambda i: (i, 0)),
        ],
        out_specs=pl.BlockSpec((BB, _FEAT), lambda i: (i, 0)),
        out_shape=jax.ShapeDtypeStruct((B, _FEAT), jnp.float32),
    )(in0, in1)


# speedup vs baseline: 22.0475x; 2.4119x over previous
import jax
import jax.numpy as jnp
from jax.experimental import pallas as pl

_NUM_SEG = 8
_EXTENT = 64
_FEAT = _NUM_SEG * _EXTENT


def _conv_kernel(x0_ref, x1_ref, o_ref):
    x0 = x0_ref[...]
    x1 = x1_ref[...]
    x1r = jnp.roll(x1, _EXTENT, axis=1)
    acc = None
    for i in range(_NUM_SEG):
        seg = x0[:, i * _EXTENT:(i + 1) * _EXTENT]
        tiled = jnp.concatenate([seg] * _NUM_SEG, axis=1)
        base = x1 if i % 2 == 0 else x1r
        shift = (i // 2) * 2 * _EXTENT
        rolled = jnp.roll(base, shift, axis=1) if shift else base
        term = tiled * rolled
        acc = term if acc is None else acc + term
    o_ref[...] = acc


def kernel(in0, in1):
    B = in0.shape[0]
    BB = 1024
    return pl.pallas_call(
        _conv_kernel,
        grid=(B // BB,),
        in_specs=[
            pl.BlockSpec((BB, _FEAT), lambda i: (i, 0)),
            pl.BlockSpec((BB, _FEAT), lambda i: (i, 0)),
        ],
        out_specs=pl.BlockSpec((BB, _FEAT), lambda i: (i, 0)),
        out_shape=jax.ShapeDtypeStruct((B, _FEAT), jnp.float32),
    )(in0, in1)
